# baseline (speedup 1.0000x reference)
import functools

import jax
import jax.numpy as jnp
from jax import lax
from jax.experimental import pallas as pl
from jax.experimental.pallas import tpu as pltpu
from jax.experimental.pallas import tpu_sc as plsc

_LAMBDA_D_PHN = 0.1
_LAMBDA_T_PHN = 1.0
_MARGIN = 0.2
_P = 40
_F = 256
_B = 8192
_NBLK = 4
_N = 32768

_NW = 32
_TW = _N // _NW
_C = 128
_NCHUNK = _TW // _C


_ACC = (_P + 1) * _F


def _sc_segsum_body(ft_hbm, phn_hbm, gt_hbm, out_hbm,
                    chunk_v, phn_v, gt_v, idx_v, acc_v):
    cid = lax.axis_index("c")
    sid = lax.axis_index("s")
    wid = sid * 2 + cid
    base = wid * _TW
    iota16 = lax.broadcasted_iota(jnp.int32, (16,), 0)

    def zr(k, carry):
        acc_v[pl.ds(k * 16, 16)] = jnp.zeros((16,), jnp.float32)
        return carry
    lax.fori_loop(0, _ACC // 16, zr, 0)

    pltpu.sync_copy(phn_hbm.at[pl.ds(base, _TW)], phn_v)
    pltpu.sync_copy(gt_hbm.at[pl.ds(base, _TW)], gt_v)

    def ib(k, c2):
        sl = pl.ds(k * 16, 16)
        idx_v[sl] = jnp.where(gt_v[sl] > 0, phn_v[sl], _P)
        return c2
    lax.fori_loop(0, _TW // 16, ib, 0)

    def chunk_body(ci, carry):
        pltpu.sync_copy(
            ft_hbm.at[pl.ds((base + ci * _C) * _F, _C * _F)], chunk_v)

        def tok_body(tl, c2):
            row = plsc.load_gather(
                idx_v, [jnp.full((16,), ci * _C + tl, jnp.int32)])
            rowbase = row * _F + iota16
            for j in range(_F // 16):
                v = chunk_v[pl.ds(tl * _F + j * 16, 16)]
                plsc.addupdate_scatter(acc_v, [rowbase + j * 16], v)
            return c2
        lax.fori_loop(0, _C, tok_body, 0)
        return carry

    lax.fori_loop(0, _NCHUNK, chunk_body, 0)

    pltpu.sync_copy(acc_v, out_hbm.at[pl.ds(wid * _ACC, _ACC)])


def _sc_segsum(ft_flat, phn, gt):
    mesh = plsc.VectorSubcoreMesh(core_axis_name="c", subcore_axis_name="s")
    return pl.kernel(
        _sc_segsum_body,
        mesh=mesh,
        compiler_params=pltpu.CompilerParams(needs_layout_passes=False),
        out_type=jax.ShapeDtypeStruct((_NW * _ACC,), jnp.float32),
        scratch_types=[
            pltpu.VMEM((_C * _F,), jnp.float32),
            pltpu.VMEM((_TW,), jnp.int32),
            pltpu.VMEM((_TW,), jnp.int32),
            pltpu.VMEM((_TW,), jnp.int32),
            pltpu.VMEM((_ACC,), jnp.float32),
        ],
    )(ft_flat, phn, gt)


def _tc_main(gtr_ref, phnr_ref, f_ref,
             loss_ref, sums_o, cnt2_o,
             fbuf, p_s, k_s, ts_s, tc_s):
    i = pl.program_id(0)

    @pl.when(i == 0)
    def _():
        sums_o[...] = jnp.zeros_like(sums_o)
        cnt2_o[...] = jnp.zeros_like(cnt2_o)
        ts_s[0] = 0.0
        tc_s[0] = 0.0

    @pl.when(i < _NBLK)
    def _():
        phn_r = phnr_ref[0]
        gt_r = gtr_ref[0]
        fb = f_ref[...].astype(jnp.bfloat16)
        iota_c = lax.broadcasted_iota(jnp.int32, (_P, 1), 0)
        cond = (phn_r == iota_c) & (gt_r > 0)
        ohT = jnp.where(cond, 1.0, 0.0)
        ohT_bf = ohT.astype(jnp.bfloat16)
        dn = (((1,), (0,)), ((), ()))
        sums_o[...] += lax.dot_general(ohT_bf, fb, dn,
                                       preferred_element_type=jnp.float32)
        hind_r = jnp.where(gt_r == 2, 1.0, 0.0)
        cn = jnp.sum(ohT, axis=1, keepdims=True)
        ch = jnp.sum(ohT * hind_r, axis=1, keepdims=True)
        cnt2_o[...] += jnp.concatenate([cn, ch], axis=1)
        fbuf[pl.ds(i * _B, _B), :] = fb

    @pl.when(i == _NBLK)
    def _():
        iota_c = lax.broadcasted_iota(jnp.int32, (_P, 1), 0)
        cn_c = cnt2_o[:, 0:1]
        ch_c = cnt2_o[:, 1:2]
        present_norm = cn_c > 0.0
        skip = present_norm & (~(ch_c > 0.0))
        any_skip = jnp.sum(jnp.where(skip, 1.0, 0.0)) > 0.0
        has_nonskip = jnp.sum(
            jnp.where(present_norm & (~skip), 1.0, 0.0)) > 0.0
        keep_if_skip = jnp.where(
            ~((iota_c == 1) | ((iota_c == 0) & has_nonskip)), 1.0, 0.0)
        keep_c = jnp.where(any_skip, keep_if_skip, 1.0)
        k_s[0] = jnp.where(any_skip & has_nonskip, 1.0, 0.0)
        k_s[1] = jnp.where(any_skip, 1.0, 0.0)

        counts_raw = keep_c * cn_c
        present_c = counts_raw > 0.0
        counts_c = jnp.where(present_c, counts_raw, 1.0)
        n_u = jnp.sum(jnp.where(present_c, 1.0, 0.0))

        def norm_rows(x):
            n = jnp.sqrt(jnp.sum(x * x, axis=1, keepdims=True))
            return x / jnp.maximum(n, 1e-12)

        center = norm_rows(keep_c * sums_o[...] / counts_c)
        p = norm_rows(center)

        r = lax.broadcasted_iota(jnp.int32, (_P, _P), 0)
        c = lax.broadcasted_iota(jnp.int32, (_P, _P), 1)
        eye = jnp.where(r == c, 1.0, 0.0)
        xx = jnp.sum(p * p, axis=1, keepdims=True)
        xx_r = lax.dot_general(xx, eye, (((0,), (0,)), ((), ())),
                               preferred_element_type=jnp.float32)
        dn = (((1,), (1,)), ((), ()))
        gram = lax.dot_general(p, p, dn,
                               preferred_element_type=jnp.float32)
        dist = jnp.sqrt(jnp.maximum(xx + xx_r - 2.0 * gram, 1e-12))
        present_r = lax.dot_general(
            jnp.where(present_c, 1.0, 0.0), eye, (((0,), (0,)), ((), ())),
            preferred_element_type=jnp.float32) > 0.0
        pair_mask = present_c & present_r & (c > r)
        n_pairs = n_u * (n_u - 1.0) / 2.0
        k_s[2] = jnp.sum(jnp.where(pair_mask, dist, 0.0)) / n_pairs
        p_s[...] = p.astype(jnp.bfloat16)

    @pl.when(i > _NBLK)
    def _():
        j = i - _NBLK - 1
        phn_r = phnr_ref[0]
        gt_r = gtr_ref[0]
        fb = fbuf[pl.ds(j * _B, _B), :]
        iota_c = lax.broadcasted_iota(jnp.int32, (_P, 1), 0)
        ohT = phn_r == iota_c

        ones_f = jnp.full((1, _F), jnp.bfloat16(1))
        dn = (((1,), (1,)), ((), ()))
        sq = lax.dot_general(ones_f, fb * fb, dn,
                             preferred_element_type=jnp.float32)
        dotsT = lax.dot_general(p_s[...], fb, dn,
                                preferred_element_type=jnp.float32)
        dotg = jnp.sum(jnp.where(ohT, dotsT, 0.0), axis=0,
                       keepdims=True)

        s = jnp.maximum(jnp.sqrt(sq), 1e-12)
        tight = sq / (s * s) + 1.0 - 2.0 * dotg / s
        killed = (jnp.where(phn_r == 0, k_s[0], 0.0)
                  + jnp.where(phn_r == 1, k_s[1], 0.0))
        tmask = (gt_r > 0) & (killed < 0.5) & (tight > 0.0)
        ordinal = 2.0 - gt_r.astype(jnp.float32) + _MARGIN
        tv = jnp.sqrt(jnp.maximum(tight, 0.0)) * ordinal
        ts_s[0] += jnp.sum(jnp.where(tmask, tv, 0.0))
        tc_s[0] += jnp.sum(jnp.where(tmask, 1.0, 0.0))

        @pl.when(i == 2 * _NBLK)
        def _():
            tight_m = ts_s[0] / tc_s[0]
            loss_ref[...] = jnp.full((1, 1), 0.0) + (
                _LAMBDA_T_PHN * tight_m - _LAMBDA_D_PHN * k_s[2])


def _tc_clap(sums_ref, cnt2_ref, scpart_ref, clap_ref):
    iota_c = lax.broadcasted_iota(jnp.int32, (_P, 1), 0)
    cn_c = cnt2_ref[:, 0:1]
    ch_c = cnt2_ref[:, 1:2]
    present_norm = cn_c > 0.0
    skip = present_norm & (~(ch_c > 0.0))
    any_skip = jnp.sum(jnp.where(skip, 1.0, 0.0)) > 0.0
    has_nonskip = jnp.sum(jnp.where(present_norm & (~skip), 1.0, 0.0)) > 0.0
    keep_if_skip = jnp.where(
        ~((iota_c == 1) | ((iota_c == 0) & has_nonskip)), 1.0, 0.0)
    keep_c = jnp.where(any_skip, keep_if_skip, 1.0)

    counts_raw = keep_c * cn_c
    present_c = counts_raw > 0.0
    counts_c = jnp.where(present_c, counts_raw, 1.0)
    n_u = jnp.sum(jnp.where(present_c, 1.0, 0.0))

    def norm_rows(x):
        n = jnp.sqrt(jnp.sum(x * x, axis=1, keepdims=True))
        return x / jnp.maximum(n, 1e-12)

    sums_t = scpart_ref[0, 0:_P, :]
    for w in range(1, _NW):
        sums_t = sums_t + scpart_ref[w, 0:_P, :]
    center = norm_rows(keep_c * sums_ref[...] / counts_c)
    center_t = norm_rows(keep_c * sums_t / counts_c)

    dn = (((1,), (1,)), ((), ()))
    logits = lax.dot_general(center, center_t, dn,
                             preferred_element_type=jnp.float32)
    r = lax.broadcasted_iota(jnp.int32, (_P, _P), 0)
    c = lax.broadcasted_iota(jnp.int32, (_P, _P), 1)
    eye = jnp.where(r == c, 1.0, 0.0)
    present_r = lax.dot_general(
        jnp.where(present_c, 1.0, 0.0), eye, (((0,), (0,)), ((), ())),
        preferred_element_type=jnp.float32) > 0.0
    logits = jnp.where(present_r, logits, jnp.float32(-jnp.inf))
    m = jnp.max(logits, axis=1, keepdims=True)
    lse = jnp.log(jnp.sum(jnp.exp(logits - m), axis=1, keepdims=True)) + m
    cos = logits - lse
    diag = jnp.sum(jnp.where(r == c, cos, 0.0), axis=0, keepdims=True)
    loss_a = -jnp.sum(jnp.where(present_r, diag, 0.0)) / n_u
    clap_ref[...] = jnp.full((1, 1), 0.0) + loss_a


def kernel(features, features_text, gt, phn_id):
    f32 = jnp.float32

    fs = features.reshape(_N, _F)
    fts = features_text.reshape(_N, _F)
    gt_flat = gt.reshape(_N).astype(jnp.int32)
    phn_flat = phn_id.reshape(_N).astype(jnp.int32)
    gtr = gt.reshape(_NBLK, 1, _B).astype(jnp.int32)
    phnr = phn_id.reshape(_NBLK, 1, _B).astype(jnp.int32)

    scpart = _sc_segsum(fts.reshape(_N * _F), phn_flat, gt_flat)
    scpart = scpart.reshape(_NW, _P + 1, _F)

    def tok3_map(i):
        j = jnp.where(i < _NBLK, i,
                      jnp.where(i == _NBLK, _NBLK - 1, i - _NBLK - 1))
        return (j, 0, 0)

    def feat_map(i):
        return (jnp.minimum(i, _NBLK - 1), 0)

    tok3_spec = pl.BlockSpec((1, 1, _B), tok3_map)
    feat_spec = pl.BlockSpec((_B, _F), feat_map)
    one_spec = pl.BlockSpec((1, 1), lambda i: (0, 0))
    acc_spec = pl.BlockSpec((_P, _F), lambda i: (0, 0))
    cnt_spec = pl.BlockSpec((_P, 2), lambda i: (0, 0))

    loss_oe, sums, cnt2 = pl.pallas_call(
        _tc_main,
        grid=(2 * _NBLK + 1,),
        in_specs=[tok3_spec, tok3_spec, feat_spec],
        out_specs=[one_spec, acc_spec, cnt_spec],
        out_shape=[
            jax.ShapeDtypeStruct((1, 1), f32),
            jax.ShapeDtypeStruct((_P, _F), f32),
            jax.ShapeDtypeStruct((_P, 2), f32),
        ],
        scratch_shapes=[
            pltpu.VMEM((_N, _F), jnp.bfloat16),
            pltpu.VMEM((_P, _F), jnp.bfloat16),
            pltpu.SMEM((3,), f32),
            pltpu.SMEM((1,), f32),
            pltpu.SMEM((1,), f32),
        ],
    )(gtr, phnr, fs)

    clap = pl.pallas_call(
        _tc_clap,
        out_shape=jax.ShapeDtypeStruct((1, 1), f32),
    )(sums, cnt2, scpart)

    return loss_oe.reshape(()), clap.reshape(())


# speedup vs baseline: 1.0618x; 1.0618x over previous
import functools

import jax
import jax.numpy as jnp
from jax import lax
from jax.experimental import pallas as pl
from jax.experimental.pallas import tpu as pltpu
from jax.experimental.pallas import tpu_sc as plsc

_LAMBDA_D_PHN = 0.1
_LAMBDA_T_PHN = 1.0
_MARGIN = 0.2
_P = 40
_F = 256
_B = 8192
_NBLK = 4
_N = 32768

_NW = 32
_TW = _N // _NW
_C = 128
_NCHUNK = _TW // _C


_ACC = (_P + 1) * _F


def _sc_segsum_body(ft_hbm, phn_hbm, gt_hbm, out_hbm,
                    buf0_v, buf1_v, phn_v, gt_v, idx_v, acc_v,
                    sem0, sem1):
    cid = lax.axis_index("c")
    sid = lax.axis_index("s")
    wid = sid * 2 + cid
    base = wid * _TW
    iota16 = lax.broadcasted_iota(jnp.int32, (16,), 0)

    def zr(k, carry):
        acc_v[pl.ds(k * 16, 16)] = jnp.zeros((16,), jnp.float32)
        return carry
    lax.fori_loop(0, _ACC // 16, zr, 0)

    pltpu.sync_copy(phn_hbm.at[pl.ds(base, _TW)], phn_v)
    pltpu.sync_copy(gt_hbm.at[pl.ds(base, _TW)], gt_v)

    def ib(k, c2):
        sl = pl.ds(k * 16, 16)
        idx_v[sl] = jnp.where(gt_v[sl] > 0, phn_v[sl], _P) * _F
        return c2
    lax.fori_loop(0, _TW // 16, ib, 0)

    bufs = (buf0_v, buf1_v)
    sems = (sem0, sem1)

    def start(ci):
        return pltpu.async_copy(
            ft_hbm.at[pl.ds((base + ci * _C) * _F, _C * _F)],
            bufs[ci % 2], sems[ci % 2])

    pending = start(0)
    for ci in range(_NCHUNK):
        pending.wait()
        if ci + 1 < _NCHUNK:
            pending = start(ci + 1)
        buf = bufs[ci % 2]

        def grp_body(g, c2):
            idx16 = idx_v[pl.ds(ci * _C + g * 16, 16)]
            for l in range(16):
                off = jnp.sum(jnp.where(iota16 == l, idx16, 0))
                for j in range(_F // 16):
                    v = buf[pl.ds((g * 16 + l) * _F + j * 16, 16)]
                    plsc.addupdate(acc_v.at[pl.ds(off + j * 16, 16)], v)
            return c2
        lax.fori_loop(0, _C // 16, grp_body, 0)

    pltpu.sync_copy(acc_v, out_hbm.at[pl.ds(wid * _ACC, _ACC)])


def _sc_segsum(ft_flat, phn, gt):
    mesh = plsc.VectorSubcoreMesh(core_axis_name="c", subcore_axis_name="s")
    return pl.kernel(
        _sc_segsum_body,
        mesh=mesh,
        compiler_params=pltpu.CompilerParams(needs_layout_passes=False),
        out_type=jax.ShapeDtypeStruct((_NW * _ACC,), jnp.float32),
        scratch_types=[
            pltpu.VMEM((_C * _F,), jnp.float32),
            pltpu.VMEM((_C * _F,), jnp.float32),
            pltpu.VMEM((_TW,), jnp.int32),
            pltpu.VMEM((_TW,), jnp.int32),
            pltpu.VMEM((_TW,), jnp.int32),
            pltpu.VMEM((_ACC,), jnp.float32),
            pltpu.SemaphoreType.DMA,
            pltpu.SemaphoreType.DMA,
        ],
    )(ft_flat, phn, gt)


def _tc_main(gtr_ref, phnr_ref, f_ref,
             loss_ref, sums_o, cnt2_o,
             fbuf, p_s, k_s, ts_s, tc_s):
    i = pl.program_id(0)

    @pl.when(i == 0)
    def _():
        sums_o[...] = jnp.zeros_like(sums_o)
        cnt2_o[...] = jnp.zeros_like(cnt2_o)
        ts_s[0] = 0.0
        tc_s[0] = 0.0

    @pl.when(i < _NBLK)
    def _():
        phn_r = phnr_ref[0]
        gt_r = gtr_ref[0]
        fb = f_ref[...].astype(jnp.bfloat16)
        iota_c = lax.broadcasted_iota(jnp.int32, (_P, 1), 0)
        cond = (phn_r == iota_c) & (gt_r > 0)
        ohT = jnp.where(cond, 1.0, 0.0)
        ohT_bf = ohT.astype(jnp.bfloat16)
        dn = (((1,), (0,)), ((), ()))
        sums_o[...] += lax.dot_general(ohT_bf, fb, dn,
                                       preferred_element_type=jnp.float32)
        hind_r = jnp.where(gt_r == 2, 1.0, 0.0)
        cn = jnp.sum(ohT, axis=1, keepdims=True)
        ch = jnp.sum(ohT * hind_r, axis=1, keepdims=True)
        cnt2_o[...] += jnp.concatenate([cn, ch], axis=1)
        fbuf[pl.ds(i * _B, _B), :] = fb

    @pl.when(i == _NBLK)
    def _():
        iota_c = lax.broadcasted_iota(jnp.int32, (_P, 1), 0)
        cn_c = cnt2_o[:, 0:1]
        ch_c = cnt2_o[:, 1:2]
        present_norm = cn_c > 0.0
        skip = present_norm & (~(ch_c > 0.0))
        any_skip = jnp.sum(jnp.where(skip, 1.0, 0.0)) > 0.0
        has_nonskip = jnp.sum(
            jnp.where(present_norm & (~skip), 1.0, 0.0)) > 0.0
        keep_if_skip = jnp.where(
            ~((iota_c == 1) | ((iota_c == 0) & has_nonskip)), 1.0, 0.0)
        keep_c = jnp.where(any_skip, keep_if_skip, 1.0)
        k_s[0] = jnp.where(any_skip & has_nonskip, 1.0, 0.0)
        k_s[1] = jnp.where(any_skip, 1.0, 0.0)

        counts_raw = keep_c * cn_c
        present_c = counts_raw > 0.0
        counts_c = jnp.where(present_c, counts_raw, 1.0)
        n_u = jnp.sum(jnp.where(present_c, 1.0, 0.0))

        def norm_rows(x):
            n = jnp.sqrt(jnp.sum(x * x, axis=1, keepdims=True))
            return x / jnp.maximum(n, 1e-12)

        center = norm_rows(keep_c * sums_o[...] / counts_c)
        p = norm_rows(center)

        r = lax.broadcasted_iota(jnp.int32, (_P, _P), 0)
        c = lax.broadcasted_iota(jnp.int32, (_P, _P), 1)
        eye = jnp.where(r == c, 1.0, 0.0)
        xx = jnp.sum(p * p, axis=1, keepdims=True)
        xx_r = lax.dot_general(xx, eye, (((0,), (0,)), ((), ())),
                               preferred_element_type=jnp.float32)
        dn = (((1,), (1,)), ((), ()))
        gram = lax.dot_general(p, p, dn,
                               preferred_element_type=jnp.float32)
        dist = jnp.sqrt(jnp.maximum(xx + xx_r - 2.0 * gram, 1e-12))
        present_r = lax.dot_general(
            jnp.where(present_c, 1.0, 0.0), eye, (((0,), (0,)), ((), ())),
            preferred_element_type=jnp.float32) > 0.0
        pair_mask = present_c & present_r & (c > r)
        n_pairs = n_u * (n_u - 1.0) / 2.0
        k_s[2] = jnp.sum(jnp.where(pair_mask, dist, 0.0)) / n_pairs
        p_s[...] = p.astype(jnp.bfloat16)

    @pl.when(i > _NBLK)
    def _():
        j = i - _NBLK - 1
        phn_r = phnr_ref[0]
        gt_r = gtr_ref[0]
        fb = fbuf[pl.ds(j * _B, _B), :]
        iota_c = lax.broadcasted_iota(jnp.int32, (_P, 1), 0)
        ohT = phn_r == iota_c

        ones_f = jnp.full((1, _F), jnp.bfloat16(1))
        dn = (((1,), (1,)), ((), ()))
        sq = lax.dot_general(ones_f, fb * fb, dn,
                             preferred_element_type=jnp.float32)
        dotsT = lax.dot_general(p_s[...], fb, dn,
                                preferred_element_type=jnp.float32)
        dotg = jnp.sum(jnp.where(ohT, dotsT, 0.0), axis=0,
                       keepdims=True)

        s = jnp.maximum(jnp.sqrt(sq), 1e-12)
        tight = sq / (s * s) + 1.0 - 2.0 * dotg / s
        killed = (jnp.where(phn_r == 0, k_s[0], 0.0)
                  + jnp.where(phn_r == 1, k_s[1], 0.0))
        tmask = (gt_r > 0) & (killed < 0.5) & (tight > 0.0)
        ordinal = 2.0 - gt_r.astype(jnp.float32) + _MARGIN
        tv = jnp.sqrt(jnp.maximum(tight, 0.0)) * ordinal
        ts_s[0] += jnp.sum(jnp.where(tmask, tv, 0.0))
        tc_s[0] += jnp.sum(jnp.where(tmask, 1.0, 0.0))

        @pl.when(i == 2 * _NBLK)
        def _():
            tight_m = ts_s[0] / tc_s[0]
            loss_ref[...] = jnp.full((1, 1), 0.0) + (
                _LAMBDA_T_PHN * tight_m - _LAMBDA_D_PHN * k_s[2])


def _tc_clap(sums_ref, cnt2_ref, scpart_ref, clap_ref):
    iota_c = lax.broadcasted_iota(jnp.int32, (_P, 1), 0)
    cn_c = cnt2_ref[:, 0:1]
    ch_c = cnt2_ref[:, 1:2]
    present_norm = cn_c > 0.0
    skip = present_norm & (~(ch_c > 0.0))
    any_skip = jnp.sum(jnp.where(skip, 1.0, 0.0)) > 0.0
    has_nonskip = jnp.sum(jnp.where(present_norm & (~skip), 1.0, 0.0)) > 0.0
    keep_if_skip = jnp.where(
        ~((iota_c == 1) | ((iota_c == 0) & has_nonskip)), 1.0, 0.0)
    keep_c = jnp.where(any_skip, keep_if_skip, 1.0)

    counts_raw = keep_c * cn_c
    present_c = counts_raw > 0.0
    counts_c = jnp.where(present_c, counts_raw, 1.0)
    n_u = jnp.sum(jnp.where(present_c, 1.0, 0.0))

    def norm_rows(x):
        n = jnp.sqrt(jnp.sum(x * x, axis=1, keepdims=True))
        return x / jnp.maximum(n, 1e-12)

    sums_t = scpart_ref[0, 0:_P, :]
    for w in range(1, _NW):
        sums_t = sums_t + scpart_ref[w, 0:_P, :]
    center = norm_rows(keep_c * sums_ref[...] / counts_c)
    center_t = norm_rows(keep_c * sums_t / counts_c)

    dn = (((1,), (1,)), ((), ()))
    logits = lax.dot_general(center, center_t, dn,
                             preferred_element_type=jnp.float32)
    r = lax.broadcasted_iota(jnp.int32, (_P, _P), 0)
    c = lax.broadcasted_iota(jnp.int32, (_P, _P), 1)
    eye = jnp.where(r == c, 1.0, 0.0)
    present_r = lax.dot_general(
        jnp.where(present_c, 1.0, 0.0), eye, (((0,), (0,)), ((), ())),
        preferred_element_type=jnp.float32) > 0.0
    logits = jnp.where(present_r, logits, jnp.float32(-jnp.inf))
    m = jnp.max(logits, axis=1, keepdims=True)
    lse = jnp.log(jnp.sum(jnp.exp(logits - m), axis=1, keepdims=True)) + m
    cos = logits - lse
    diag = jnp.sum(jnp.where(r == c, cos, 0.0), axis=0, keepdims=True)
    loss_a = -jnp.sum(jnp.where(present_r, diag, 0.0)) / n_u
    clap_ref[...] = jnp.full((1, 1), 0.0) + loss_a


def kernel(features, features_text, gt, phn_id):
    f32 = jnp.float32

    fs = features.reshape(_N, _F)
    fts = features_text.reshape(_N, _F)
    gt_flat = gt.reshape(_N).astype(jnp.int32)
    phn_flat = phn_id.reshape(_N).astype(jnp.int32)
    gtr = gt.reshape(_NBLK, 1, _B).astype(jnp.int32)
    phnr = phn_id.reshape(_NBLK, 1, _B).astype(jnp.int32)

    scpart = _sc_segsum(fts.reshape(_N * _F), phn_flat, gt_flat)
    scpart = scpart.reshape(_NW, _P + 1, _F)

    def tok3_map(i):
        j = jnp.where(i < _NBLK, i,
                      jnp.where(i == _NBLK, _NBLK - 1, i - _NBLK - 1))
        return (j, 0, 0)

    def feat_map(i):
        return (jnp.minimum(i, _NBLK - 1), 0)

    tok3_spec = pl.BlockSpec((1, 1, _B), tok3_map)
    feat_spec = pl.BlockSpec((_B, _F), feat_map)
    one_spec = pl.BlockSpec((1, 1), lambda i: (0, 0))
    acc_spec = pl.BlockSpec((_P, _F), lambda i: (0, 0))
    cnt_spec = pl.BlockSpec((_P, 2), lambda i: (0, 0))

    loss_oe, sums, cnt2 = pl.pallas_call(
        _tc_main,
        grid=(2 * _NBLK + 1,),
        in_specs=[tok3_spec, tok3_spec, feat_spec],
        out_specs=[one_spec, acc_spec, cnt_spec],
        out_shape=[
            jax.ShapeDtypeStruct((1, 1), f32),
            jax.ShapeDtypeStruct((_P, _F), f32),
            jax.ShapeDtypeStruct((_P, 2), f32),
        ],
        scratch_shapes=[
            pltpu.VMEM((_N, _F), jnp.bfloat16),
            pltpu.VMEM((_P, _F), jnp.bfloat16),
            pltpu.SMEM((3,), f32),
            pltpu.SMEM((1,), f32),
            pltpu.SMEM((1,), f32),
        ],
    )(gtr, phnr, fs)

    clap = pl.pallas_call(
        _tc_clap,
        out_shape=jax.ShapeDtypeStruct((1, 1), f32),
    )(sums, cnt2, scpart)

    return loss_oe.reshape(()), clap.reshape(())


# speedup vs baseline: 3.9337x; 3.7047x over previous
import jax
import jax.numpy as jnp
from jax import lax
from jax.experimental import pallas as pl
from jax.experimental.pallas import tpu as pltpu

_LAMBDA_D_PHN = 0.1
_LAMBDA_T_PHN = 1.0
_MARGIN = 0.2
_P = 40
_F = 256
_B = 8192
_NBLK = 4


def _fused_kernel(gtr_ref, phnr_ref, f_ref, ft_ref,
                  loss_ref, clap_ref,
                  fbuf, sums_s, sums_t_s, cnt2_s,
                  p_s, center_s, ckp_s, k_s, ts_s, tc_s):
    i = pl.program_id(0)

    @pl.when(i == 0)
    def _():
        sums_s[...] = jnp.zeros_like(sums_s)
        sums_t_s[...] = jnp.zeros_like(sums_t_s)
        cnt2_s[...] = jnp.zeros_like(cnt2_s)
        ts_s[0] = 0.0
        tc_s[0] = 0.0

    @pl.when(i < _NBLK)
    def _():
        phn_r = phnr_ref[0]
        gt_r = gtr_ref[0]
        fb = f_ref[...].astype(jnp.bfloat16)
        iota_c = lax.broadcasted_iota(jnp.int32, (_P, 1), 0)
        cond = (phn_r == iota_c) & (gt_r > 0)
        ohT = jnp.where(cond, 1.0, 0.0)
        ohT_bf = ohT.astype(jnp.bfloat16)
        dn = (((1,), (0,)), ((), ()))
        sums_s[...] += lax.dot_general(ohT_bf, fb, dn,
                                       preferred_element_type=jnp.float32)
        hind_r = jnp.where(gt_r == 2, 1.0, 0.0)
        cn = jnp.sum(ohT, axis=1, keepdims=True)
        ch = jnp.sum(ohT * hind_r, axis=1, keepdims=True)
        cnt2_s[...] += jnp.concatenate([cn, ch], axis=1)
        fbuf[pl.ds(i * _B, _B), :] = fb

    @pl.when(i == _NBLK)
    def _():
        r = lax.broadcasted_iota(jnp.int32, (_P, _P), 0)
        c = lax.broadcasted_iota(jnp.int32, (_P, _P), 1)
        eye = jnp.where(r == c, 1.0, 0.0)

        def col_to_row(v):
            return lax.dot_general(v, eye, (((0,), (0,)), ((), ())),
                                   preferred_element_type=jnp.float32)

        iota_c = lax.broadcasted_iota(jnp.int32, (_P, 1), 0)
        cn_c = cnt2_s[:, 0:1]
        ch_c = cnt2_s[:, 1:2]
        present_norm = cn_c > 0.0
        skip = present_norm & (~(ch_c > 0.0))
        any_skip = jnp.sum(jnp.where(skip, 1.0, 0.0)) > 0.0
        has_nonskip = jnp.sum(
            jnp.where(present_norm & (~skip), 1.0, 0.0)) > 0.0
        keep_if_skip = jnp.where(
            ~((iota_c == 1) | ((iota_c == 0) & has_nonskip)), 1.0, 0.0)
        keep_c = jnp.where(any_skip, keep_if_skip, 1.0)
        k_s[0] = jnp.where(any_skip & has_nonskip, 1.0, 0.0)
        k_s[1] = jnp.where(any_skip, 1.0, 0.0)

        counts_raw = keep_c * cn_c
        present_c = counts_raw > 0.0
        counts_c = jnp.where(present_c, counts_raw, 1.0)
        n_u = jnp.sum(jnp.where(present_c, 1.0, 0.0))

        def norm_rows(x):
            n = jnp.sqrt(jnp.sum(x * x, axis=1, keepdims=True))
            return x / jnp.maximum(n, 1e-12)

        center = norm_rows(keep_c * sums_s[...] / counts_c)
        center_s[...] = center
        ckp_s[:, 0:1] = counts_c
        ckp_s[:, 1:2] = keep_c
        ckp_s[:, 2:3] = jnp.where(present_c, 1.0, 0.0)
        k_s[3] = n_u

        dn = (((1,), (1,)), ((), ()))
        present_r = col_to_row(jnp.where(present_c, 1.0, 0.0)) > 0.0
        p = norm_rows(center)
        xx = jnp.sum(p * p, axis=1, keepdims=True)
        xx_r = col_to_row(xx)
        gram = lax.dot_general(p, p, dn,
                               preferred_element_type=jnp.float32)
        dist = jnp.sqrt(jnp.maximum(xx + xx_r - 2.0 * gram, 1e-12))
        pair_mask = present_c & present_r & (c > r)
        n_pairs = n_u * (n_u - 1.0) / 2.0
        k_s[2] = jnp.sum(jnp.where(pair_mask, dist, 0.0)) / n_pairs
        p_s[...] = p.astype(jnp.bfloat16)

    @pl.when(i > _NBLK)
    def _():
        j = i - _NBLK - 1
        phn_r = phnr_ref[0]
        gt_r = gtr_ref[0]
        fb = fbuf[pl.ds(j * _B, _B), :]
        iota_c = lax.broadcasted_iota(jnp.int32, (_P, 1), 0)
        ohT = phn_r == iota_c
        ohTf = jnp.where(ohT, 1.0, 0.0)
        gt_mask = jnp.where(gt_r > 0, 1.0, 0.0)
        ohT_bf = (ohTf * gt_mask).astype(jnp.bfloat16)

        dn0 = (((1,), (0,)), ((), ()))
        sums_t_s[...] += lax.dot_general(
            ohT_bf, ft_ref[...].astype(jnp.bfloat16), dn0,
            preferred_element_type=jnp.float32)

        ones_f = jnp.full((1, _F), jnp.bfloat16(1))
        dn = (((1,), (1,)), ((), ()))
        sq = lax.dot_general(ones_f, fb * fb, dn,
                             preferred_element_type=jnp.float32)
        dotsT = lax.dot_general(p_s[...], fb, dn,
                                preferred_element_type=jnp.float32)
        dotg = jnp.sum(ohTf * dotsT, axis=0, keepdims=True)

        s = jnp.maximum(jnp.sqrt(sq), 1e-12)
        tight = sq / (s * s) + 1.0 - 2.0 * dotg / s
        killed = (jnp.where(phn_r == 0, k_s[0], 0.0)
                  + jnp.where(phn_r == 1, k_s[1], 0.0))
        tmask = (gt_r > 0) & (killed < 0.5) & (tight > 0.0)
        ordinal = 2.0 - gt_r.astype(jnp.float32) + _MARGIN
        tv = jnp.sqrt(jnp.maximum(tight, 0.0)) * ordinal
        ts_s[0] += jnp.sum(jnp.where(tmask, tv, 0.0))
        tc_s[0] += jnp.sum(jnp.where(tmask, 1.0, 0.0))

        @pl.when(i == 2 * _NBLK)
        def _():
            tight_m = ts_s[0] / tc_s[0]
            loss_ref[...] = jnp.full((1, 1), 0.0) + (
                _LAMBDA_T_PHN * tight_m - _LAMBDA_D_PHN * k_s[2])

            r = lax.broadcasted_iota(jnp.int32, (_P, _P), 0)
            c = lax.broadcasted_iota(jnp.int32, (_P, _P), 1)
            eye = jnp.where(r == c, 1.0, 0.0)
            counts_c = ckp_s[:, 0:1]
            keep_c = ckp_s[:, 1:2]
            present_f = ckp_s[:, 2:3]

            def norm_rows(x):
                n = jnp.sqrt(jnp.sum(x * x, axis=1, keepdims=True))
                return x / jnp.maximum(n, 1e-12)

            center_t = norm_rows(keep_c * sums_t_s[...] / counts_c)
            dnr = (((1,), (1,)), ((), ()))
            logits = lax.dot_general(center_s[...], center_t, dnr,
                                     preferred_element_type=jnp.float32)
            present_r = lax.dot_general(
                present_f, eye, (((0,), (0,)), ((), ())),
                preferred_element_type=jnp.float32) > 0.0
            logits = jnp.where(present_r, logits, jnp.float32(-jnp.inf))
            m = jnp.max(logits, axis=1, keepdims=True)
            lse = jnp.log(jnp.sum(jnp.exp(logits - m), axis=1,
                                  keepdims=True)) + m
            cos = logits - lse
            diag = jnp.sum(jnp.where(r == c, cos, 0.0),
                           axis=0, keepdims=True)
            loss_a = -jnp.sum(jnp.where(present_r, diag, 0.0)) / k_s[3]
            clap_ref[...] = jnp.full((1, 1), 0.0) + loss_a


def kernel(features, features_text, gt, phn_id):
    N = features.shape[0] * features.shape[1]
    f32 = jnp.float32

    fs = features.reshape(N, _F)
    fts = features_text.reshape(N, _F)
    gtr = gt.reshape(_NBLK, 1, _B).astype(jnp.int32)
    phnr = phn_id.reshape(_NBLK, 1, _B).astype(jnp.int32)

    def tok3_map(i):
        j = jnp.where(i < _NBLK, i,
                      jnp.where(i == _NBLK, _NBLK - 1, i - _NBLK - 1))
        return (j, 0, 0)

    def feat_map(i):
        return (jnp.minimum(i, _NBLK - 1), 0)

    def ft_map(i):
        return (jnp.where(i > _NBLK, i - _NBLK - 1, 0), 0)

    tok3_spec = pl.BlockSpec((1, 1, _B), tok3_map)
    feat_spec = pl.BlockSpec((_B, _F), feat_map)
    ft_spec = pl.BlockSpec((_B, _F), ft_map)
    one_spec = pl.BlockSpec((1, 1), lambda i: (0, 0))

    loss_oe, clap = pl.pallas_call(
        _fused_kernel,
        grid=(2 * _NBLK + 1,),
        in_specs=[tok3_spec, tok3_spec, feat_spec, ft_spec],
        out_specs=[one_spec, one_spec],
        out_shape=[
            jax.ShapeDtypeStruct((1, 1), f32),
            jax.ShapeDtypeStruct((1, 1), f32),
        ],
        scratch_shapes=[
            pltpu.VMEM((N, _F), jnp.bfloat16),
            pltpu.VMEM((_P, _F), f32),
            pltpu.VMEM((_P, _F), f32),
            pltpu.VMEM((_P, 2), f32),
            pltpu.VMEM((_P, _F), jnp.bfloat16),
            pltpu.VMEM((_P, _F), f32),
            pltpu.VMEM((_P, 4), f32),
            pltpu.SMEM((4,), f32),
            pltpu.SMEM((1,), f32),
            pltpu.SMEM((1,), f32),
        ],
    )(gtr, phnr, fs, fts)

    return loss_oe.reshape(()), clap.reshape(())
